# Initial kernel scaffold; baseline (speedup 1.0000x reference)
import jax
import jax.numpy as jnp
from jax.experimental import pallas as pl
from jax.experimental.pallas import tpu as pltpu

HID = 1024
FFD = 2048
NE = 8
T = 2048
TB = 256


def _router_body(x_ref, w_ref, scores_ref):
    x = x_ref[...]
    w = w_ref[...]
    logits = jnp.dot(x, w, preferred_element_type=jnp.float32)
    m = jnp.max(logits, axis=1, keepdims=True)
    lane = jax.lax.broadcasted_iota(jnp.int32, logits.shape, 1)
    mi = jnp.min(jnp.where(logits == m, lane, NE), axis=1, keepdims=True)
    onehot = lane == mi
    scores_ref[...] = jnp.where(onehot, jax.nn.sigmoid(logits), 0.0)


def _moe_body(x_ref, wg_ref, wu_ref, wd_ref, sc_ref, out_ref):
    j = pl.program_id(1)
    xb = x_ref[...].astype(jnp.bfloat16)
    g = jnp.dot(xb, wg_ref[0], preferred_element_type=jnp.float32)
    u = jnp.dot(xb, wu_ref[0], preferred_element_type=jnp.float32)
    h = (g * jax.nn.sigmoid(g) * u).astype(jnp.bfloat16)
    y = jnp.dot(h, wd_ref[0], preferred_element_type=jnp.float32)
    sc = sc_ref[...]
    lane = jax.lax.broadcasted_iota(jnp.int32, sc.shape, 1)
    sel = jnp.sum(jnp.where(lane == j, sc, 0.0), axis=1, keepdims=True)

    @pl.when(j == 0)
    def _init():
        out_ref[...] = jnp.zeros_like(out_ref)

    out_ref[...] += y * sel


def kernel(hidden_states, router_w, shared_gate, shared_up, shared_down,
           expert_gate, expert_up, expert_down):
    x = hidden_states.reshape(-1, HID)

    scores_masked = pl.pallas_call(
        _router_body,
        out_shape=jax.ShapeDtypeStruct((T, NE), jnp.float32),
    )(x, router_w)

    wg = jnp.concatenate([shared_gate[None], expert_gate], axis=0).astype(jnp.bfloat16)
    wu = jnp.concatenate([shared_up[None], expert_up], axis=0).astype(jnp.bfloat16)
    wd = jnp.concatenate([shared_down[None], expert_down], axis=0).astype(jnp.bfloat16)
    scales = jnp.concatenate([jnp.ones((T, 1), jnp.float32), scores_masked], axis=1)

    out = pl.pallas_call(
        _moe_body,
        grid=(T // TB, NE + 1),
        in_specs=[
            pl.BlockSpec((TB, HID), lambda tb, j: (tb, 0)),
            pl.BlockSpec((1, HID, FFD), lambda tb, j: (j, 0, 0)),
            pl.BlockSpec((1, HID, FFD), lambda tb, j: (j, 0, 0)),
            pl.BlockSpec((1, FFD, HID), lambda tb, j: (j, 0, 0)),
            pl.BlockSpec((TB, NE + 1), lambda tb, j: (tb, 0)),
        ],
        out_specs=pl.BlockSpec((TB, HID), lambda tb, j: (tb, 0)),
        out_shape=jax.ShapeDtypeStruct((T, HID), jnp.float32),
    )(x, wg, wu, wd, scales)

    return (out, scores_masked.T)


# speedup vs baseline: 1.4447x; 1.4447x over previous
import jax
import jax.numpy as jnp
from jax.experimental import pallas as pl
from jax.experimental.pallas import tpu as pltpu
from jax.experimental.pallas import tpu_sc as plsc

HID = 1024
FFD = 2048
FH = FFD // 2
NE = 8
T = 2048
TB = 256
GBLK = T // TB + NE
S = GBLK * TB
CH = 256
IPT = HID // CH
NI = T * IPT
SCWIN = 128


def _route_body(x_ref, w_ref, scores_ref, idx_ref, bexp_ref, nact_ref):
    x = x_ref[...]
    logits = jnp.dot(x, w_ref[...], preferred_element_type=jnp.float32)
    m = jnp.max(logits, axis=1, keepdims=True)
    lane = jax.lax.broadcasted_iota(jnp.int32, (T, NE), 1)
    mi = jnp.min(jnp.where(logits == m, lane, NE), axis=1, keepdims=True)
    onehot = lane == mi
    scores_ref[...] = jnp.where(onehot, jax.nn.sigmoid(logits), 0.0)

    oh = onehot.astype(jnp.int32)
    c = oh
    sh = 1
    while sh < T:
        c = c + jnp.pad(c, ((sh, 0), (0, 0)))[:T]
        sh *= 2
    rank_excl = c - oh
    counts = c[T - 1:T, :]

    nblk = (counts + TB - 1) // TB
    e = nblk
    sh = 1
    while sh < NE:
        e = e + jnp.pad(e, ((0, 0), (sh, 0)))[:, :NE]
        sh *= 2
    cum_incl = e
    off = (cum_incl - nblk) * TB

    rank_sel = jnp.sum(jnp.where(onehot, rank_excl, 0), axis=1, keepdims=True)
    off_sel = jnp.sum(jnp.where(onehot, jnp.broadcast_to(off, (T, NE)), 0),
                      axis=1, keepdims=True)
    dest = off_sel + rank_sel
    j_io = jax.lax.broadcasted_iota(jnp.int32, (T, IPT), 1)
    idx_ref[...] = dest * IPT + j_io

    nact = cum_incl[0:1, NE - 1:NE]
    nact_ref[...] = nact
    g_io = jax.lax.broadcasted_iota(jnp.int32, (GBLK, NE), 0)
    g_eff = jnp.minimum(g_io, nact - 1)
    bexp_ref[...] = jnp.sum((g_eff >= cum_incl).astype(jnp.int32),
                            axis=1, keepdims=True)


def _route(x, router_w):
    return pl.pallas_call(
        _route_body,
        out_shape=(
            jax.ShapeDtypeStruct((T, NE), jnp.float32),
            jax.ShapeDtypeStruct((T, IPT), jnp.int32),
            jax.ShapeDtypeStruct((GBLK, 1), jnp.int32),
            jax.ShapeDtypeStruct((1, 1), jnp.int32),
        ),
    )(x, router_w)


def _gmm_body(bexp_ref, nact_ref, x_ref, wg_ref, wu_ref, wd_ref, y_ref):
    g = pl.program_id(0)
    f = pl.program_id(1)

    @pl.when(g < nact_ref[0])
    def _():
        xb = x_ref[...].astype(jnp.bfloat16)
        wg = wg_ref[0].astype(jnp.bfloat16)
        wu = wu_ref[0].astype(jnp.bfloat16)
        wd = wd_ref[0].astype(jnp.bfloat16)
        gg = jnp.dot(xb, wg, preferred_element_type=jnp.float32)
        uu = jnp.dot(xb, wu, preferred_element_type=jnp.float32)
        h = (gg * jax.nn.sigmoid(gg) * uu).astype(jnp.bfloat16)
        y = jnp.dot(h, wd, preferred_element_type=jnp.float32)

        @pl.when(f == 0)
        def _():
            y_ref[...] = y

        @pl.when(f != 0)
        def _():
            y_ref[...] += y


def _gmm(bexp1d, nact1d, x_pad, eg, eu, ed):
    grid_spec = pltpu.PrefetchScalarGridSpec(
        num_scalar_prefetch=2,
        grid=(GBLK, 2),
        in_specs=[
            pl.BlockSpec((TB, HID), lambda g, f, bexp, nact: (g, 0)),
            pl.BlockSpec((1, HID, FH), lambda g, f, bexp, nact: (bexp[g], 0, f)),
            pl.BlockSpec((1, HID, FH), lambda g, f, bexp, nact: (bexp[g], 0, f)),
            pl.BlockSpec((1, FH, HID), lambda g, f, bexp, nact: (bexp[g], f, 0)),
        ],
        out_specs=pl.BlockSpec((TB, HID), lambda g, f, bexp, nact: (g, 0)),
    )
    return pl.pallas_call(
        _gmm_body,
        grid_spec=grid_spec,
        out_shape=jax.ShapeDtypeStruct((S, HID), jnp.float32),
    )(bexp1d, nact1d, x_pad, eg, eu, ed)


def _vector_mesh():
    return plsc.VectorSubcoreMesh(core_axis_name="core",
                                  subcore_axis_name="subcore")


def _sc_dispatch(x2, idx2d):
    @pl.kernel(out_type=jax.ShapeDtypeStruct((S * IPT, CH), jnp.float32),
               mesh=_vector_mesh())
    def k(x_hbm, i_hbm, o_hbm):
        def body(x_vmem, i_vmem):
            pltpu.sync_copy(x_vmem, o_hbm.at[i_vmem.at[0]])

        pltpu.emit_pipeline(
            body,
            grid=(NI // SCWIN,),
            in_specs=[pl.BlockSpec((SCWIN, CH), lambda i: (i, 0)),
                      pl.BlockSpec((1, SCWIN), lambda i: (0, i))],
            out_specs=[],
            core_axis_name=("core", "subcore"),
            dimension_semantics=(pltpu.PARALLEL,),
        )(x_hbm, i_hbm)

    return k(x2, idx2d)


def _sc_combine(y2, idx2d):
    @pl.kernel(out_type=jax.ShapeDtypeStruct((NI, CH), jnp.float32),
               mesh=_vector_mesh())
    def k(y_hbm, i_hbm, o_hbm):
        def body(i_vmem, o_vmem):
            pltpu.sync_copy(y_hbm.at[i_vmem.at[0]], o_vmem)

        pltpu.emit_pipeline(
            body,
            grid=(NI // SCWIN,),
            in_specs=[pl.BlockSpec((1, SCWIN), lambda i: (0, i))],
            out_specs=[pl.BlockSpec((SCWIN, CH), lambda i: (i, 0))],
            core_axis_name=("core", "subcore"),
            dimension_semantics=(pltpu.PARALLEL,),
        )(i_hbm, o_hbm)

    return k(y2, idx2d)


def _shared_body(x_ref, wg_ref, wu_ref, wd_ref, out_ref):
    f = pl.program_id(1)
    xb = x_ref[...].astype(jnp.bfloat16)
    wg = wg_ref[...].astype(jnp.bfloat16)
    wu = wu_ref[...].astype(jnp.bfloat16)
    wd = wd_ref[...].astype(jnp.bfloat16)
    gg = jnp.dot(xb, wg, preferred_element_type=jnp.float32)
    uu = jnp.dot(xb, wu, preferred_element_type=jnp.float32)
    h = (gg * jax.nn.sigmoid(gg) * uu).astype(jnp.bfloat16)
    y = jnp.dot(h, wd, preferred_element_type=jnp.float32)

    @pl.when(f == 0)
    def _():
        out_ref[...] = y

    @pl.when(f != 0)
    def _():
        out_ref[...] += y


def _shared(x, sg, su, sd):
    return pl.pallas_call(
        _shared_body,
        grid=(T // TB, 2),
        in_specs=[
            pl.BlockSpec((TB, HID), lambda i, f: (i, 0)),
            pl.BlockSpec((HID, FH), lambda i, f: (0, f)),
            pl.BlockSpec((HID, FH), lambda i, f: (0, f)),
            pl.BlockSpec((FH, HID), lambda i, f: (f, 0)),
        ],
        out_specs=pl.BlockSpec((TB, HID), lambda i, f: (i, 0)),
        out_shape=jax.ShapeDtypeStruct((T, HID), jnp.float32),
    )(x, sg, su, sd)


def _add_body(ysh_ref, yt_ref, sc_ref, out_ref):
    s = jnp.sum(sc_ref[...], axis=1, keepdims=True)
    out_ref[...] = ysh_ref[...] + yt_ref[...] * s


def _add(y_sh, y_tok, scores):
    return pl.pallas_call(
        _add_body,
        grid=(T // TB,),
        in_specs=[
            pl.BlockSpec((TB, HID), lambda i: (i, 0)),
            pl.BlockSpec((TB, HID), lambda i: (i, 0)),
            pl.BlockSpec((TB, NE), lambda i: (i, 0)),
        ],
        out_specs=pl.BlockSpec((TB, HID), lambda i: (i, 0)),
        out_shape=jax.ShapeDtypeStruct((T, HID), jnp.float32),
    )(y_sh, y_tok, scores)


def kernel(hidden_states, router_w, shared_gate, shared_up, shared_down,
           expert_gate, expert_up, expert_down):
    x = hidden_states.reshape(-1, HID)

    scores, idx, bexp, nact = _route(x, router_w)
    idx2d = idx.reshape(1, NI)
    bexp1d = bexp.reshape(GBLK)
    nact1d = nact.reshape(1)

    x_pad = _sc_dispatch(x.reshape(T * IPT, CH), idx2d).reshape(S, HID)

    y_pad = _gmm(bexp1d, nact1d, x_pad, expert_gate, expert_up, expert_down)

    y_sh = _shared(x, shared_gate, shared_up, shared_down)

    y_tok = _sc_combine(y_pad.reshape(S * IPT, CH), idx2d).reshape(T, HID)

    out = _add(y_sh, y_tok, scores)

    return (out, scores.T)


# speedup vs baseline: 1.7988x; 1.2452x over previous
import jax
import jax.numpy as jnp
from jax.experimental import pallas as pl
from jax.experimental.pallas import tpu as pltpu
from jax.experimental.pallas import tpu_sc as plsc

HID = 1024
FFD = 2048
FH = FFD // 2
NE = 8
T = 2048
TB = 256
GBLK = T // TB + NE
S = GBLK * TB
CH = 256
IPT = HID // CH
NI = T * IPT
SCWIN = 128


def _route_body(x_ref, w_ref, scores_ref, idx_ref, bexp_ref, nact_ref):
    x = x_ref[...]
    logits = jnp.dot(x, w_ref[...], preferred_element_type=jnp.float32)
    m = jnp.max(logits, axis=1, keepdims=True)
    lane = jax.lax.broadcasted_iota(jnp.int32, (T, NE), 1)
    mi = jnp.min(jnp.where(logits == m, lane, NE), axis=1, keepdims=True)
    onehot = lane == mi
    scores_ref[...] = jnp.where(onehot, jax.nn.sigmoid(logits), 0.0)

    oh = onehot.astype(jnp.int32)
    c = oh
    sh = 1
    while sh < T:
        c = c + jnp.pad(c, ((sh, 0), (0, 0)))[:T]
        sh *= 2
    rank_excl = c - oh
    counts = c[T - 1:T, :]

    nblk = (counts + TB - 1) // TB
    e = nblk
    sh = 1
    while sh < NE:
        e = e + jnp.pad(e, ((0, 0), (sh, 0)))[:, :NE]
        sh *= 2
    cum_incl = e
    off = (cum_incl - nblk) * TB

    rank_sel = jnp.sum(jnp.where(onehot, rank_excl, 0), axis=1, keepdims=True)
    off_sel = jnp.sum(jnp.where(onehot, jnp.broadcast_to(off, (T, NE)), 0),
                      axis=1, keepdims=True)
    dest = off_sel + rank_sel
    j_io = jax.lax.broadcasted_iota(jnp.int32, (T, IPT), 1)
    idx_ref[...] = dest * IPT + j_io

    nact = cum_incl[0:1, NE - 1:NE]
    nact_ref[...] = nact
    g_io = jax.lax.broadcasted_iota(jnp.int32, (GBLK, NE), 0)
    g_eff = jnp.minimum(g_io, nact - 1)
    bexp_ref[...] = jnp.sum((g_eff >= cum_incl).astype(jnp.int32),
                            axis=1, keepdims=True)


def _route(x, router_w):
    return pl.pallas_call(
        _route_body,
        out_shape=(
            jax.ShapeDtypeStruct((T, NE), jnp.float32),
            jax.ShapeDtypeStruct((T, IPT), jnp.int32),
            jax.ShapeDtypeStruct((GBLK, 1), jnp.int32),
            jax.ShapeDtypeStruct((1, 1), jnp.int32),
        ),
    )(x, router_w)


def _gmm_body(bexp_ref, nact_ref, x_ref, wg_ref, wu_ref, wd_ref, y_ref):
    g = pl.program_id(0)

    @pl.when(g < nact_ref[0])
    def _():
        xb = x_ref[...].astype(jnp.bfloat16)
        wg = wg_ref[0].astype(jnp.bfloat16)
        wu = wu_ref[0].astype(jnp.bfloat16)
        wd = wd_ref[0].astype(jnp.bfloat16)
        gg = jnp.dot(xb, wg, preferred_element_type=jnp.float32)
        uu = jnp.dot(xb, wu, preferred_element_type=jnp.float32)
        h = (gg * jax.nn.sigmoid(gg) * uu).astype(jnp.bfloat16)
        y_ref[...] = jnp.dot(h, wd, preferred_element_type=jnp.float32)


def _gmm(bexp1d, nact1d, x_pad, eg, eu, ed):
    grid_spec = pltpu.PrefetchScalarGridSpec(
        num_scalar_prefetch=2,
        grid=(GBLK,),
        in_specs=[
            pl.BlockSpec((TB, HID), lambda g, bexp, nact: (g, 0)),
            pl.BlockSpec((1, HID, FFD), lambda g, bexp, nact: (bexp[g], 0, 0)),
            pl.BlockSpec((1, HID, FFD), lambda g, bexp, nact: (bexp[g], 0, 0)),
            pl.BlockSpec((1, FFD, HID), lambda g, bexp, nact: (bexp[g], 0, 0)),
        ],
        out_specs=pl.BlockSpec((TB, HID), lambda g, bexp, nact: (g, 0)),
    )
    return pl.pallas_call(
        _gmm_body,
        grid_spec=grid_spec,
        out_shape=jax.ShapeDtypeStruct((S, HID), jnp.float32),
        compiler_params=pltpu.CompilerParams(
            vmem_limit_bytes=120 * 1024 * 1024),
    )(bexp1d, nact1d, x_pad, eg, eu, ed)


def _vector_mesh():
    return plsc.VectorSubcoreMesh(core_axis_name="core",
                                  subcore_axis_name="subcore")


def _sc_dispatch(x2, idx2d):
    @pl.kernel(out_type=jax.ShapeDtypeStruct((S * IPT, CH), jnp.float32),
               mesh=_vector_mesh())
    def k(x_hbm, i_hbm, o_hbm):
        def body(x_vmem, i_vmem):
            pltpu.sync_copy(x_vmem, o_hbm.at[i_vmem.at[0]])

        pltpu.emit_pipeline(
            body,
            grid=(NI // SCWIN,),
            in_specs=[pl.BlockSpec((SCWIN, CH), lambda i: (i, 0)),
                      pl.BlockSpec((1, SCWIN), lambda i: (0, i))],
            out_specs=[],
            core_axis_name=("core", "subcore"),
            dimension_semantics=(pltpu.PARALLEL,),
        )(x_hbm, i_hbm)

    return k(x2, idx2d)


def _sc_combine(y2, idx2d):
    @pl.kernel(out_type=jax.ShapeDtypeStruct((NI, CH), jnp.float32),
               mesh=_vector_mesh())
    def k(y_hbm, i_hbm, o_hbm):
        def body(i_vmem, o_vmem):
            pltpu.sync_copy(y_hbm.at[i_vmem.at[0]], o_vmem)

        pltpu.emit_pipeline(
            body,
            grid=(NI // SCWIN,),
            in_specs=[pl.BlockSpec((1, SCWIN), lambda i: (0, i))],
            out_specs=[pl.BlockSpec((SCWIN, CH), lambda i: (i, 0))],
            core_axis_name=("core", "subcore"),
            dimension_semantics=(pltpu.PARALLEL,),
        )(i_hbm, o_hbm)

    return k(y2, idx2d)


def _shared_body(x_ref, wg_ref, wu_ref, wd_ref, out_ref):
    xb = x_ref[...].astype(jnp.bfloat16)
    wg = wg_ref[...].astype(jnp.bfloat16)
    wu = wu_ref[...].astype(jnp.bfloat16)
    wd = wd_ref[...].astype(jnp.bfloat16)
    gg = jnp.dot(xb, wg, preferred_element_type=jnp.float32)
    uu = jnp.dot(xb, wu, preferred_element_type=jnp.float32)
    h = (gg * jax.nn.sigmoid(gg) * uu).astype(jnp.bfloat16)
    out_ref[...] = jnp.dot(h, wd, preferred_element_type=jnp.float32)


def _shared(x, sg, su, sd):
    return pl.pallas_call(
        _shared_body,
        grid=(T // TB,),
        in_specs=[
            pl.BlockSpec((TB, HID), lambda i: (i, 0)),
            pl.BlockSpec((HID, FFD), lambda i: (0, 0)),
            pl.BlockSpec((HID, FFD), lambda i: (0, 0)),
            pl.BlockSpec((FFD, HID), lambda i: (0, 0)),
        ],
        out_specs=pl.BlockSpec((TB, HID), lambda i: (i, 0)),
        out_shape=jax.ShapeDtypeStruct((T, HID), jnp.float32),
        compiler_params=pltpu.CompilerParams(
            vmem_limit_bytes=120 * 1024 * 1024),
    )(x, sg, su, sd)


def _add_body(ysh_ref, yt_ref, sc_ref, out_ref):
    s = jnp.sum(sc_ref[...], axis=1, keepdims=True)
    out_ref[...] = ysh_ref[...] + yt_ref[...] * s


def _add(y_sh, y_tok, scores):
    return pl.pallas_call(
        _add_body,
        grid=(T // TB,),
        in_specs=[
            pl.BlockSpec((TB, HID), lambda i: (i, 0)),
            pl.BlockSpec((TB, HID), lambda i: (i, 0)),
            pl.BlockSpec((TB, NE), lambda i: (i, 0)),
        ],
        out_specs=pl.BlockSpec((TB, HID), lambda i: (i, 0)),
        out_shape=jax.ShapeDtypeStruct((T, HID), jnp.float32),
    )(y_sh, y_tok, scores)


def kernel(hidden_states, router_w, shared_gate, shared_up, shared_down,
           expert_gate, expert_up, expert_down):
    x = hidden_states.reshape(-1, HID)

    scores, idx, bexp, nact = _route(x, router_w)
    idx2d = idx.reshape(1, NI)
    bexp1d = bexp.reshape(GBLK)
    nact1d = nact.reshape(1)

    x_pad = _sc_dispatch(x.reshape(T * IPT, CH), idx2d).reshape(S, HID)

    y_pad = _gmm(bexp1d, nact1d, x_pad, expert_gate, expert_up, expert_down)

    y_sh = _shared(x, shared_gate, shared_up, shared_down)

    y_tok = _sc_combine(y_pad.reshape(S * IPT, CH), idx2d).reshape(T, HID)

    out = _add(y_sh, y_tok, scores)

    return (out, scores.T)


# speedup vs baseline: 1.8509x; 1.0290x over previous
import jax
import jax.numpy as jnp
from jax.experimental import pallas as pl
from jax.experimental.pallas import tpu as pltpu
from jax.experimental.pallas import tpu_sc as plsc

HID = 1024
FFD = 2048
FH = FFD // 2
NE = 8
T = 2048
TB = 256
GBLK = T // TB + NE
S = GBLK * TB
CH = 256
IPT = HID // CH
NI = T * IPT
SCWIN = 128


def _route_body(x_ref, w_ref, scores_ref, idx_ref, bexp_ref, nact_ref):
    x = x_ref[...]
    logits = jnp.dot(x, w_ref[...], preferred_element_type=jnp.float32)
    m = jnp.max(logits, axis=1, keepdims=True)
    lane = jax.lax.broadcasted_iota(jnp.int32, (T, NE), 1)
    mi = jnp.min(jnp.where(logits == m, lane, NE), axis=1, keepdims=True)
    onehot = lane == mi
    scores_ref[...] = jnp.where(onehot, jax.nn.sigmoid(logits), 0.0)

    oh = onehot.astype(jnp.int32)
    c = oh
    sh = 1
    while sh < T:
        c = c + jnp.pad(c, ((sh, 0), (0, 0)))[:T]
        sh *= 2
    rank_excl = c - oh
    counts = c[T - 1:T, :]

    nblk = (counts + TB - 1) // TB
    e = nblk
    sh = 1
    while sh < NE:
        e = e + jnp.pad(e, ((0, 0), (sh, 0)))[:, :NE]
        sh *= 2
    cum_incl = e
    off = (cum_incl - nblk) * TB

    rank_sel = jnp.sum(jnp.where(onehot, rank_excl, 0), axis=1, keepdims=True)
    off_sel = jnp.sum(jnp.where(onehot, jnp.broadcast_to(off, (T, NE)), 0),
                      axis=1, keepdims=True)
    dest = off_sel + rank_sel
    j_io = jax.lax.broadcasted_iota(jnp.int32, (T, IPT), 1)
    idx_ref[...] = dest * IPT + j_io

    nact = cum_incl[0:1, NE - 1:NE]
    nact_ref[...] = nact
    g_io = jax.lax.broadcasted_iota(jnp.int32, (GBLK, NE), 0)
    g_eff = jnp.minimum(g_io, nact - 1)
    bexp_ref[...] = jnp.sum((g_eff >= cum_incl).astype(jnp.int32),
                            axis=1, keepdims=True)


def _route(x, router_w):
    return pl.pallas_call(
        _route_body,
        out_shape=(
            jax.ShapeDtypeStruct((T, NE), jnp.float32),
            jax.ShapeDtypeStruct((T, IPT), jnp.int32),
            jax.ShapeDtypeStruct((GBLK, 1), jnp.int32),
            jax.ShapeDtypeStruct((1, 1), jnp.int32),
        ),
    )(x, router_w)


def _gmm_body(bexp_ref, nact_ref, x_ref, wg_ref, wu_ref, wd_ref, y_ref):
    g = pl.program_id(0)

    @pl.when(g < nact_ref[0])
    def _():
        xb = x_ref[...].astype(jnp.bfloat16)
        wg = wg_ref[0].astype(jnp.bfloat16)
        wu = wu_ref[0].astype(jnp.bfloat16)
        wd = wd_ref[0].astype(jnp.bfloat16)
        gg = jnp.dot(xb, wg, preferred_element_type=jnp.float32)
        uu = jnp.dot(xb, wu, preferred_element_type=jnp.float32)
        h = (gg * jax.nn.sigmoid(gg) * uu).astype(jnp.bfloat16)
        y_ref[...] = jnp.dot(h, wd, preferred_element_type=jnp.float32)


def _gmm(bexp1d, nact1d, x_pad, eg, eu, ed):
    grid_spec = pltpu.PrefetchScalarGridSpec(
        num_scalar_prefetch=2,
        grid=(GBLK,),
        in_specs=[
            pl.BlockSpec((TB, HID), lambda g, bexp, nact: (g, 0)),
            pl.BlockSpec((1, HID, FFD), lambda g, bexp, nact: (bexp[g], 0, 0)),
            pl.BlockSpec((1, HID, FFD), lambda g, bexp, nact: (bexp[g], 0, 0)),
            pl.BlockSpec((1, FFD, HID), lambda g, bexp, nact: (bexp[g], 0, 0)),
        ],
        out_specs=pl.BlockSpec((TB, HID), lambda g, bexp, nact: (g, 0)),
    )
    return pl.pallas_call(
        _gmm_body,
        grid_spec=grid_spec,
        out_shape=jax.ShapeDtypeStruct((S, HID), jnp.float32),
        compiler_params=pltpu.CompilerParams(
            vmem_limit_bytes=120 * 1024 * 1024),
    )(bexp1d, nact1d, x_pad, eg, eu, ed)


def _vector_mesh():
    return plsc.VectorSubcoreMesh(core_axis_name="core",
                                  subcore_axis_name="subcore")


def _sc_dispatch(x2, idx2d):
    @pl.kernel(out_type=jax.ShapeDtypeStruct((S * IPT, CH), jnp.float32),
               mesh=_vector_mesh())
    def k(x_hbm, i_hbm, o_hbm):
        def body(x_vmem, i_vmem):
            pltpu.sync_copy(x_vmem, o_hbm.at[i_vmem.at[0]])

        pltpu.emit_pipeline(
            body,
            grid=(NI // SCWIN,),
            in_specs=[pl.BlockSpec((SCWIN, CH), lambda i: (i, 0)),
                      pl.BlockSpec((1, SCWIN), lambda i: (0, i))],
            out_specs=[],
            core_axis_name=("core", "subcore"),
            dimension_semantics=(pltpu.PARALLEL,),
        )(x_hbm, i_hbm)

    return k(x2, idx2d)


def _sc_combine(y2, idx2d):
    @pl.kernel(out_type=jax.ShapeDtypeStruct((NI, CH), jnp.float32),
               mesh=_vector_mesh())
    def k(y_hbm, i_hbm, o_hbm):
        def body(i_vmem, o_vmem):
            pltpu.sync_copy(y_hbm.at[i_vmem.at[0]], o_vmem)

        pltpu.emit_pipeline(
            body,
            grid=(NI // SCWIN,),
            in_specs=[pl.BlockSpec((1, SCWIN), lambda i: (0, i))],
            out_specs=[pl.BlockSpec((SCWIN, CH), lambda i: (i, 0))],
            core_axis_name=("core", "subcore"),
            dimension_semantics=(pltpu.PARALLEL,),
        )(i_hbm, o_hbm)

    return k(y2, idx2d)


def _shared_add_body(x_ref, wg_ref, wu_ref, wd_ref, yt_ref, sc_ref, out_ref):
    xb = x_ref[...].astype(jnp.bfloat16)
    wg = wg_ref[...].astype(jnp.bfloat16)
    wu = wu_ref[...].astype(jnp.bfloat16)
    wd = wd_ref[...].astype(jnp.bfloat16)
    gg = jnp.dot(xb, wg, preferred_element_type=jnp.float32)
    uu = jnp.dot(xb, wu, preferred_element_type=jnp.float32)
    h = (gg * jax.nn.sigmoid(gg) * uu).astype(jnp.bfloat16)
    y_sh = jnp.dot(h, wd, preferred_element_type=jnp.float32)
    s = jnp.sum(sc_ref[...], axis=1, keepdims=True)
    out_ref[...] = y_sh + yt_ref[...] * s


def _shared_add(x, sg, su, sd, y_tok, scores):
    return pl.pallas_call(
        _shared_add_body,
        grid=(T // TB,),
        in_specs=[
            pl.BlockSpec((TB, HID), lambda i: (i, 0)),
            pl.BlockSpec((HID, FFD), lambda i: (0, 0)),
            pl.BlockSpec((HID, FFD), lambda i: (0, 0)),
            pl.BlockSpec((FFD, HID), lambda i: (0, 0)),
            pl.BlockSpec((TB, HID), lambda i: (i, 0)),
            pl.BlockSpec((TB, NE), lambda i: (i, 0)),
        ],
        out_specs=pl.BlockSpec((TB, HID), lambda i: (i, 0)),
        out_shape=jax.ShapeDtypeStruct((T, HID), jnp.float32),
        compiler_params=pltpu.CompilerParams(
            vmem_limit_bytes=120 * 1024 * 1024),
    )(x, sg, su, sd, y_tok, scores)


def kernel(hidden_states, router_w, shared_gate, shared_up, shared_down,
           expert_gate, expert_up, expert_down):
    x = hidden_states.reshape(-1, HID)

    scores, idx, bexp, nact = _route(x, router_w)
    idx2d = idx.reshape(1, NI)
    bexp1d = bexp.reshape(GBLK)
    nact1d = nact.reshape(1)

    x_pad = _sc_dispatch(x.reshape(T * IPT, CH), idx2d).reshape(S, HID)

    y_pad = _gmm(bexp1d, nact1d, x_pad, expert_gate, expert_up, expert_down)

    y_tok = _sc_combine(y_pad.reshape(S * IPT, CH), idx2d).reshape(T, HID)

    out = _shared_add(x, shared_gate, shared_up, shared_down, y_tok, scores)

    return (out, scores.T)


# speedup vs baseline: 1.8617x; 1.0058x over previous
import jax
import jax.numpy as jnp
from jax.experimental import pallas as pl
from jax.experimental.pallas import tpu as pltpu
from jax.experimental.pallas import tpu_sc as plsc

HID = 1024
FFD = 2048
FH = FFD // 2
NE = 8
T = 2048
TB = 256
GBLK = T // TB + NE
S = GBLK * TB
CH = 256
IPT = HID // CH
NI = T * IPT
SCWIN = 128


def _route_body(x_ref, w_ref, scores_ref, idx_ref, bexp_ref, nact_ref,
                xb16_ref):
    x = x_ref[...]
    xb16_ref[...] = x.astype(jnp.bfloat16)
    logits = jnp.dot(x, w_ref[...], preferred_element_type=jnp.float32)
    m = jnp.max(logits, axis=1, keepdims=True)
    lane = jax.lax.broadcasted_iota(jnp.int32, (T, NE), 1)
    mi = jnp.min(jnp.where(logits == m, lane, NE), axis=1, keepdims=True)
    onehot = lane == mi
    scores_ref[...] = jnp.where(onehot, jax.nn.sigmoid(logits), 0.0)

    oh = onehot.astype(jnp.int32)
    c = oh
    sh = 1
    while sh < T:
        c = c + jnp.pad(c, ((sh, 0), (0, 0)))[:T]
        sh *= 2
    rank_excl = c - oh
    counts = c[T - 1:T, :]

    nblk = (counts + TB - 1) // TB
    e = nblk
    sh = 1
    while sh < NE:
        e = e + jnp.pad(e, ((0, 0), (sh, 0)))[:, :NE]
        sh *= 2
    cum_incl = e
    off = (cum_incl - nblk) * TB

    rank_sel = jnp.sum(jnp.where(onehot, rank_excl, 0), axis=1, keepdims=True)
    off_sel = jnp.sum(jnp.where(onehot, jnp.broadcast_to(off, (T, NE)), 0),
                      axis=1, keepdims=True)
    dest = off_sel + rank_sel
    j_io = jax.lax.broadcasted_iota(jnp.int32, (T, IPT), 1)
    idx_ref[...] = dest * IPT + j_io

    nact = cum_incl[0:1, NE - 1:NE]
    nact_ref[...] = nact
    g_io = jax.lax.broadcasted_iota(jnp.int32, (GBLK, NE), 0)
    g_eff = jnp.minimum(g_io, nact - 1)
    bexp_ref[...] = jnp.sum((g_eff >= cum_incl).astype(jnp.int32),
                            axis=1, keepdims=True)


def _route(x, router_w):
    return pl.pallas_call(
        _route_body,
        out_shape=(
            jax.ShapeDtypeStruct((T, NE), jnp.float32),
            jax.ShapeDtypeStruct((T, IPT), jnp.int32),
            jax.ShapeDtypeStruct((GBLK, 1), jnp.int32),
            jax.ShapeDtypeStruct((1, 1), jnp.int32),
            jax.ShapeDtypeStruct((T, HID), jnp.bfloat16),
        ),
    )(x, router_w)


def _gmm_body(bexp_ref, nact_ref, x_ref, wg_ref, wu_ref, wd_ref, y_ref):
    g = pl.program_id(0)

    @pl.when(g < nact_ref[0])
    def _():
        xb = x_ref[...].astype(jnp.bfloat16)
        wg = wg_ref[0].astype(jnp.bfloat16)
        wu = wu_ref[0].astype(jnp.bfloat16)
        wd = wd_ref[0].astype(jnp.bfloat16)
        gg = jnp.dot(xb, wg, preferred_element_type=jnp.float32)
        uu = jnp.dot(xb, wu, preferred_element_type=jnp.float32)
        h = (gg * jax.nn.sigmoid(gg) * uu).astype(jnp.bfloat16)
        y_ref[...] = jnp.dot(h, wd, preferred_element_type=jnp.float32)


def _gmm(bexp1d, nact1d, x_pad, eg, eu, ed):
    grid_spec = pltpu.PrefetchScalarGridSpec(
        num_scalar_prefetch=2,
        grid=(GBLK,),
        in_specs=[
            pl.BlockSpec((TB, HID),
                         lambda g, bexp, nact: (jnp.minimum(g, nact[0] - 1), 0)),
            pl.BlockSpec((1, HID, FFD), lambda g, bexp, nact: (bexp[g], 0, 0)),
            pl.BlockSpec((1, HID, FFD), lambda g, bexp, nact: (bexp[g], 0, 0)),
            pl.BlockSpec((1, FFD, HID), lambda g, bexp, nact: (bexp[g], 0, 0)),
        ],
        out_specs=pl.BlockSpec(
            (TB, HID), lambda g, bexp, nact: (jnp.minimum(g, nact[0] - 1), 0)),
    )
    return pl.pallas_call(
        _gmm_body,
        grid_spec=grid_spec,
        out_shape=jax.ShapeDtypeStruct((S, HID), jnp.float32),
        compiler_params=pltpu.CompilerParams(
            vmem_limit_bytes=120 * 1024 * 1024),
    )(bexp1d, nact1d, x_pad, eg, eu, ed)


def _vector_mesh():
    return plsc.VectorSubcoreMesh(core_axis_name="core",
                                  subcore_axis_name="subcore")


def _sc_dispatch(x2, idx2d):
    @pl.kernel(out_type=jax.ShapeDtypeStruct((S * IPT, CH), jnp.float32),
               mesh=_vector_mesh())
    def k(x_hbm, i_hbm, o_hbm):
        def body(x_vmem, i_vmem):
            pltpu.sync_copy(x_vmem, o_hbm.at[i_vmem.at[0]])

        pltpu.emit_pipeline(
            body,
            grid=(NI // SCWIN,),
            in_specs=[pl.BlockSpec((SCWIN, CH), lambda i: (i, 0)),
                      pl.BlockSpec((1, SCWIN), lambda i: (0, i))],
            out_specs=[],
            core_axis_name=("core", "subcore"),
            dimension_semantics=(pltpu.PARALLEL,),
        )(x_hbm, i_hbm)

    return k(x2, idx2d)


def _sc_combine(y2, idx2d):
    @pl.kernel(out_type=jax.ShapeDtypeStruct((NI, CH), jnp.float32),
               mesh=_vector_mesh())
    def k(y_hbm, i_hbm, o_hbm):
        def body(i_vmem, o_vmem):
            pltpu.sync_copy(y_hbm.at[i_vmem.at[0]], o_vmem)

        pltpu.emit_pipeline(
            body,
            grid=(NI // SCWIN,),
            in_specs=[pl.BlockSpec((1, SCWIN), lambda i: (0, i))],
            out_specs=[pl.BlockSpec((SCWIN, CH), lambda i: (i, 0))],
            core_axis_name=("core", "subcore"),
            dimension_semantics=(pltpu.PARALLEL,),
        )(i_hbm, o_hbm)

    return k(y2, idx2d)


def _shared_add_body(x_ref, wg_ref, wu_ref, wd_ref, yt_ref, sc_ref, out_ref):
    xb = x_ref[...]
    wg = wg_ref[...].astype(jnp.bfloat16)
    wu = wu_ref[...].astype(jnp.bfloat16)
    wd = wd_ref[...].astype(jnp.bfloat16)
    gg = jnp.dot(xb, wg, preferred_element_type=jnp.float32)
    uu = jnp.dot(xb, wu, preferred_element_type=jnp.float32)
    h = (gg * jax.nn.sigmoid(gg) * uu).astype(jnp.bfloat16)
    y_sh = jnp.dot(h, wd, preferred_element_type=jnp.float32)
    s = jnp.sum(sc_ref[...], axis=1, keepdims=True)
    out_ref[...] = y_sh + yt_ref[...] * s


def _shared_add(x, sg, su, sd, y_tok, scores):
    return pl.pallas_call(
        _shared_add_body,
        grid=(T // TB,),
        in_specs=[
            pl.BlockSpec((TB, HID), lambda i: (i, 0)),
            pl.BlockSpec((HID, FFD), lambda i: (0, 0)),
            pl.BlockSpec((HID, FFD), lambda i: (0, 0)),
            pl.BlockSpec((FFD, HID), lambda i: (0, 0)),
            pl.BlockSpec((TB, HID), lambda i: (i, 0)),
            pl.BlockSpec((TB, NE), lambda i: (i, 0)),
        ],
        out_specs=pl.BlockSpec((TB, HID), lambda i: (i, 0)),
        out_shape=jax.ShapeDtypeStruct((T, HID), jnp.float32),
        compiler_params=pltpu.CompilerParams(
            vmem_limit_bytes=120 * 1024 * 1024),
    )(x, sg, su, sd, y_tok, scores)


def kernel(hidden_states, router_w, shared_gate, shared_up, shared_down,
           expert_gate, expert_up, expert_down):
    x = hidden_states.reshape(-1, HID)

    scores, idx, bexp, nact, xb16 = _route(x, router_w)
    idx2d = idx.reshape(1, NI)
    bexp1d = bexp.reshape(GBLK)
    nact1d = nact.reshape(1)

    x_pad = _sc_dispatch(x.reshape(T * IPT, CH), idx2d).reshape(S, HID)

    y_pad = _gmm(bexp1d, nact1d, x_pad, expert_gate, expert_up, expert_down)

    y_tok = _sc_combine(y_pad.reshape(S * IPT, CH), idx2d).reshape(T, HID)

    out = _shared_add(xb16, shared_gate, shared_up, shared_down, y_tok, scores)

    return (out, scores.T)


# speedup vs baseline: 1.8951x; 1.0179x over previous
import jax
import jax.numpy as jnp
from jax.experimental import pallas as pl
from jax.experimental.pallas import tpu as pltpu
from jax.experimental.pallas import tpu_sc as plsc

HID = 1024
FFD = 2048
FH = FFD // 2
NE = 8
T = 2048
TB = 128
FB = 256
GBLK = T // TB + NE
S = GBLK * TB
CH = 256
IPT = HID // CH
NI = T * IPT
SCWIN = 128


def _route_body(x_ref, w_ref, scores_ref, idx_ref, bexp_ref, nact_ref,
                xb16_ref):
    x = x_ref[...]
    xb16_ref[...] = x.astype(jnp.bfloat16)
    logits = jnp.dot(x, w_ref[...], preferred_element_type=jnp.float32)
    m = jnp.max(logits, axis=1, keepdims=True)
    lane = jax.lax.broadcasted_iota(jnp.int32, (T, NE), 1)
    mi = jnp.min(jnp.where(logits == m, lane, NE), axis=1, keepdims=True)
    onehot = lane == mi
    scores_ref[...] = jnp.where(onehot, jax.nn.sigmoid(logits), 0.0)

    oh = onehot.astype(jnp.int32)
    c = oh
    sh = 1
    while sh < T:
        c = c + jnp.pad(c, ((sh, 0), (0, 0)))[:T]
        sh *= 2
    rank_excl = c - oh
    counts = c[T - 1:T, :]

    nblk = (counts + TB - 1) // TB
    e = nblk
    sh = 1
    while sh < NE:
        e = e + jnp.pad(e, ((0, 0), (sh, 0)))[:, :NE]
        sh *= 2
    cum_incl = e
    off = (cum_incl - nblk) * TB

    rank_sel = jnp.sum(jnp.where(onehot, rank_excl, 0), axis=1, keepdims=True)
    off_sel = jnp.sum(jnp.where(onehot, jnp.broadcast_to(off, (T, NE)), 0),
                      axis=1, keepdims=True)
    dest = off_sel + rank_sel
    j_io = jax.lax.broadcasted_iota(jnp.int32, (T, IPT), 1)
    idx_ref[...] = dest * IPT + j_io

    nact = cum_incl[0:1, NE - 1:NE]
    nact_ref[...] = nact
    g_io = jax.lax.broadcasted_iota(jnp.int32, (GBLK, NE), 0)
    g_eff = jnp.minimum(g_io, nact - 1)
    bexp_ref[...] = jnp.sum((g_eff >= cum_incl).astype(jnp.int32),
                            axis=1, keepdims=True)


def _route(x, router_w):
    return pl.pallas_call(
        _route_body,
        out_shape=(
            jax.ShapeDtypeStruct((T, NE), jnp.float32),
            jax.ShapeDtypeStruct((T, IPT), jnp.int32),
            jax.ShapeDtypeStruct((GBLK, 1), jnp.int32),
            jax.ShapeDtypeStruct((1, 1), jnp.int32),
            jax.ShapeDtypeStruct((T, HID), jnp.bfloat16),
        ),
    )(x, router_w)


def _gmm_body(bexp_ref, nact_ref, x_ref, wg_ref, wu_ref, wd_ref, y_ref):
    g = pl.program_id(0)

    @pl.when(g < nact_ref[0])
    def _():
        xb = x_ref[...].astype(jnp.bfloat16)
        wg = wg_ref[0].astype(jnp.bfloat16)
        wu = wu_ref[0].astype(jnp.bfloat16)
        wd = wd_ref[0].astype(jnp.bfloat16)
        gg = jnp.dot(xb, wg, preferred_element_type=jnp.float32)
        uu = jnp.dot(xb, wu, preferred_element_type=jnp.float32)
        h = (gg * jax.nn.sigmoid(gg) * uu).astype(jnp.bfloat16)
        y_ref[...] = jnp.dot(h, wd, preferred_element_type=jnp.float32)


def _gmm(bexp1d, nact1d, x_pad, eg, eu, ed):
    grid_spec = pltpu.PrefetchScalarGridSpec(
        num_scalar_prefetch=2,
        grid=(GBLK,),
        in_specs=[
            pl.BlockSpec((TB, HID),
                         lambda g, bexp, nact: (jnp.minimum(g, nact[0] - 1), 0)),
            pl.BlockSpec((1, HID, FFD), lambda g, bexp, nact: (bexp[g], 0, 0)),
            pl.BlockSpec((1, HID, FFD), lambda g, bexp, nact: (bexp[g], 0, 0)),
            pl.BlockSpec((1, FFD, HID), lambda g, bexp, nact: (bexp[g], 0, 0)),
        ],
        out_specs=pl.BlockSpec(
            (TB, HID), lambda g, bexp, nact: (jnp.minimum(g, nact[0] - 1), 0)),
    )
    return pl.pallas_call(
        _gmm_body,
        grid_spec=grid_spec,
        out_shape=jax.ShapeDtypeStruct((S, HID), jnp.float32),
        compiler_params=pltpu.CompilerParams(
            vmem_limit_bytes=120 * 1024 * 1024),
    )(bexp1d, nact1d, x_pad, eg, eu, ed)


def _vector_mesh():
    return plsc.VectorSubcoreMesh(core_axis_name="core",
                                  subcore_axis_name="subcore")


def _sc_dispatch(x2, idx2d):
    @pl.kernel(out_type=jax.ShapeDtypeStruct((S * IPT, CH), jnp.float32),
               mesh=_vector_mesh())
    def k(x_hbm, i_hbm, o_hbm):
        def body(x_vmem, i_vmem):
            pltpu.sync_copy(x_vmem, o_hbm.at[i_vmem.at[0]])

        pltpu.emit_pipeline(
            body,
            grid=(NI // SCWIN,),
            in_specs=[pl.BlockSpec((SCWIN, CH), lambda i: (i, 0)),
                      pl.BlockSpec((1, SCWIN), lambda i: (0, i))],
            out_specs=[],
            core_axis_name=("core", "subcore"),
            dimension_semantics=(pltpu.PARALLEL,),
        )(x_hbm, i_hbm)

    return k(x2, idx2d)


def _sc_combine(y2, idx2d):
    @pl.kernel(out_type=jax.ShapeDtypeStruct((NI, CH), jnp.float32),
               mesh=_vector_mesh())
    def k(y_hbm, i_hbm, o_hbm):
        def body(i_vmem, o_vmem):
            pltpu.sync_copy(y_hbm.at[i_vmem.at[0]], o_vmem)

        pltpu.emit_pipeline(
            body,
            grid=(NI // SCWIN,),
            in_specs=[pl.BlockSpec((1, SCWIN), lambda i: (0, i))],
            out_specs=[pl.BlockSpec((SCWIN, CH), lambda i: (i, 0))],
            core_axis_name=("core", "subcore"),
            dimension_semantics=(pltpu.PARALLEL,),
        )(i_hbm, o_hbm)

    return k(y2, idx2d)


def _shared_add_body(x_ref, wg_ref, wu_ref, wd_ref, yt_ref, sc_ref, out_ref):
    xb = x_ref[...]
    wg = wg_ref[...].astype(jnp.bfloat16)
    wu = wu_ref[...].astype(jnp.bfloat16)
    wd = wd_ref[...].astype(jnp.bfloat16)
    gg = jnp.dot(xb, wg, preferred_element_type=jnp.float32)
    uu = jnp.dot(xb, wu, preferred_element_type=jnp.float32)
    h = (gg * jax.nn.sigmoid(gg) * uu).astype(jnp.bfloat16)
    y_sh = jnp.dot(h, wd, preferred_element_type=jnp.float32)
    s = jnp.sum(sc_ref[...], axis=1, keepdims=True)
    out_ref[...] = y_sh + yt_ref[...] * s


def _shared_add(x, sg, su, sd, y_tok, scores):
    return pl.pallas_call(
        _shared_add_body,
        grid=(T // FB,),
        in_specs=[
            pl.BlockSpec((FB, HID), lambda i: (i, 0)),
            pl.BlockSpec((HID, FFD), lambda i: (0, 0)),
            pl.BlockSpec((HID, FFD), lambda i: (0, 0)),
            pl.BlockSpec((FFD, HID), lambda i: (0, 0)),
            pl.BlockSpec((FB, HID), lambda i: (i, 0)),
            pl.BlockSpec((FB, NE), lambda i: (i, 0)),
        ],
        out_specs=pl.BlockSpec((FB, HID), lambda i: (i, 0)),
        out_shape=jax.ShapeDtypeStruct((T, HID), jnp.float32),
        compiler_params=pltpu.CompilerParams(
            vmem_limit_bytes=120 * 1024 * 1024),
    )(x, sg, su, sd, y_tok, scores)


def kernel(hidden_states, router_w, shared_gate, shared_up, shared_down,
           expert_gate, expert_up, expert_down):
    x = hidden_states.reshape(-1, HID)

    scores, idx, bexp, nact, xb16 = _route(x, router_w)
    idx2d = idx.reshape(1, NI)
    bexp1d = bexp.reshape(GBLK)
    nact1d = nact.reshape(1)

    x_pad = _sc_dispatch(x.reshape(T * IPT, CH), idx2d).reshape(S, HID)

    y_pad = _gmm(bexp1d, nact1d, x_pad, expert_gate, expert_up, expert_down)

    y_tok = _sc_combine(y_pad.reshape(S * IPT, CH), idx2d).reshape(T, HID)

    out = _shared_add(xb16, shared_gate, shared_up, shared_down, y_tok, scores)

    return (out, scores.T)
